# Initial kernel scaffold; baseline (speedup 1.0000x reference)
import jax
import jax.numpy as jnp
from jax.experimental import pallas as pl


def kernel(feature, edge_index, W1, b1, W2, b2, Wfc, bfc):
    raise NotImplementedError("write your pallas kernel here")


# speedup vs baseline: 3.2321x; 3.2321x over previous
import functools

import jax
import jax.numpy as jnp
from jax import lax
from jax.experimental import pallas as pl
from jax.experimental.pallas import tpu as pltpu
from jax.experimental.pallas import tpu_sc as plsc

N_PAD = 96
N_EDGE_CH = 94
COLS_PT = 200
OFFS = tuple(list(range(0, 192, 16)) + [184])


def _tanh(x):
    ax = jnp.abs(x)
    t = 1.0 - 2.0 / (jnp.exp(2.0 * ax) + 1.0)
    return jnp.sign(x) * t


def _rsqrt(d):
    bits = lax.bitcast_convert_type(d, jnp.int32)
    y = lax.bitcast_convert_type(
        jnp.int32(0x5F3759DF) - (bits >> 1), jnp.float32)
    half = 0.5 * d
    for _ in range(4):
        y = y * (1.5 - half * y * y)
    return y


def _sc_gcn(feat, row, col, params, wfc, bfc):
    mesh = plsc.VectorSubcoreMesh(core_axis_name="c", subcore_axis_name="s")

    @functools.partial(
        pl.kernel,
        mesh=mesh,
        out_type=jax.ShapeDtypeStruct((6400,), jnp.float32),
        compiler_params=pltpu.CompilerParams(
            use_tc_tiling_on_sc=False, needs_layout_passes=False),
        scratch_types=[
            pltpu.VMEM((N_PAD,), jnp.float32),
            pltpu.VMEM((1504,), jnp.int32),
            pltpu.VMEM((1504,), jnp.int32),
            pltpu.VMEM((16,), jnp.float32),
            pltpu.VMEM((94, COLS_PT), jnp.float32),
            pltpu.VMEM((COLS_PT,), jnp.float32),
            pltpu.VMEM((N_PAD,), jnp.float32),
            pltpu.VMEM((N_PAD,), jnp.float32),
            pltpu.VMEM((N_PAD,), jnp.float32),
            pltpu.VMEM((N_PAD,), jnp.float32),
            pltpu.VMEM((N_PAD,), jnp.float32),
            pltpu.VMEM((N_PAD,), jnp.float32),
            pltpu.VMEM((N_PAD,), jnp.float32),
            pltpu.VMEM((N_PAD,), jnp.float32),
            pltpu.VMEM((N_PAD,), jnp.float32),
            pltpu.SemaphoreType.DMA,
        ],
    )
    def k(feat_hbm, row_hbm, col_hbm, par_hbm, wfc_hbm, bfc_hbm, out_hbm,
          feat_v, row_v, col_v, par_v, wblk_v, obuf_v, dinv_v,
          a0, a1, a2, a3, g1_v, g2_v, agg2_v, v_v, sem):
        wid = lax.axis_index("s") * 2 + lax.axis_index("c")
        base = wid * COLS_PT

        wcp = pltpu.async_copy(wfc_hbm.at[:, pl.ds(base, COLS_PT)],
                               wblk_v, sem)
        pltpu.sync_copy(feat_hbm, feat_v)
        pltpu.sync_copy(row_hbm, row_v)
        pltpu.sync_copy(col_hbm, col_v)
        pltpu.sync_copy(par_hbm, par_v)
        pltpu.sync_copy(bfc_hbm.at[pl.ds(base, COLS_PT)], obuf_v)

        ones = jnp.ones((16,), jnp.float32)
        for i in range(N_PAD // 16):
            dinv_v[pl.ds(i * 16, 16)] = ones

        def deg_body(e, _):
            c = col_v[pl.ds(e * 16, 16)]
            plsc.addupdate_scatter(dinv_v, [c], ones)
            return 0

        lax.fori_loop(0, N_EDGE_CH, deg_body, 0)

        pv = par_v[pl.ds(0, 16)]
        w10, w11, w12, w13 = pv[0], pv[1], pv[2], pv[3]
        b10, b11, b12, b13 = pv[4], pv[5], pv[6], pv[7]
        w20, w21, w22, w23 = pv[8], pv[9], pv[10], pv[11]
        b2s = pv[12]

        for i in range(N_PAD // 16):
            sl = pl.ds(i * 16, 16)
            di = _rsqrt(dinv_v[sl])
            dinv_v[sl] = di
            g = di * feat_v[sl]
            g1_v[sl] = g
            a0[sl] = g * w10
            a1[sl] = g * w11
            a2[sl] = g * w12
            a3[sl] = g * w13

        def edge1_body(e, _):
            sl = pl.ds(e * 16, 16)
            r = row_v[sl]
            c = col_v[sl]
            g = plsc.load_gather(g1_v, [r])
            plsc.addupdate_scatter(a0, [c], g * w10)
            plsc.addupdate_scatter(a1, [c], g * w11)
            plsc.addupdate_scatter(a2, [c], g * w12)
            plsc.addupdate_scatter(a3, [c], g * w13)
            return 0

        lax.fori_loop(0, N_EDGE_CH, edge1_body, 0)

        for i in range(N_PAD // 16):
            sl = pl.ds(i * 16, 16)
            di = dinv_v[sl]
            h0 = _tanh(a0[sl] * di + b10)
            h1 = _tanh(a1[sl] * di + b11)
            h2 = _tanh(a2[sl] * di + b12)
            h3 = _tanh(a3[sl] * di + b13)
            x2 = h0 * w20 + h1 * w21 + h2 * w22 + h3 * w23
            g2 = di * x2
            g2_v[sl] = g2
            agg2_v[sl] = g2

        def edge2_body(e, _):
            sl = pl.ds(e * 16, 16)
            r = row_v[sl]
            c = col_v[sl]
            g = plsc.load_gather(g2_v, [r])
            plsc.addupdate_scatter(agg2_v, [c], g)
            return 0

        lax.fori_loop(0, N_EDGE_CH, edge2_body, 0)

        for i in range(N_PAD // 16):
            sl = pl.ds(i * 16, 16)
            v_v[sl] = _tanh(agg2_v[sl] * dinv_v[sl] + b2s)

        wcp.wait()

        def mv_outer(i, accs):
            vvec = v_v[pl.ds(i * 16, 16)]
            nb = i * 16
            for l in range(16):
                s = vvec[l]
                accs = tuple(accs[j] + s * wblk_v[nb + l, pl.ds(OFFS[j], 16)]
                             for j in range(len(OFFS)))
            return accs

        init = tuple(obuf_v[pl.ds(o, 16)] for o in OFFS)
        accs = lax.fori_loop(0, 5, mv_outer, init)
        vtail = v_v[pl.ds(80, 16)]
        for l in range(14):
            s = vtail[l]
            accs = tuple(accs[j] + s * wblk_v[80 + l, pl.ds(OFFS[j], 16)]
                         for j in range(len(OFFS)))
        for j, o in enumerate(OFFS):
            obuf_v[pl.ds(o, 16)] = accs[j]
        pltpu.sync_copy(obuf_v, out_hbm.at[pl.ds(base, COLS_PT)])

    return k(feat, row, col, params, wfc, bfc)


def kernel(feature, edge_index, W1, b1, W2, b2, Wfc, bfc):
    feat = jnp.zeros((N_PAD,), jnp.float32).at[:94].set(feature[:, 0])
    row = edge_index[0].astype(jnp.int32)
    col = edge_index[1].astype(jnp.int32)
    params = jnp.concatenate([
        W1[0], b1, W2[:, 0], b2, jnp.zeros((3,), jnp.float32)])
    return _sc_gcn(feat, row, col, params, Wfc, bfc)


# speedup vs baseline: 3.3003x; 1.0211x over previous
import functools

import jax
import jax.numpy as jnp
from jax import lax
from jax.experimental import pallas as pl
from jax.experimental.pallas import tpu as pltpu
from jax.experimental.pallas import tpu_sc as plsc

N_PAD = 96
N_EDGE_CH = 94
COLS_PT = 200
OFFS = tuple(list(range(0, 192, 16)) + [184])


def _tanh(x):
    ax = jnp.abs(x)
    t = 1.0 - 2.0 / (jnp.exp(2.0 * ax) + 1.0)
    return jnp.sign(x) * t


def _rsqrt(d):
    bits = lax.bitcast_convert_type(d, jnp.int32)
    y = lax.bitcast_convert_type(
        jnp.int32(0x5F3759DF) - (bits >> 1), jnp.float32)
    half = 0.5 * d
    for _ in range(4):
        y = y * (1.5 - half * y * y)
    return y


def _sc_gcn(feat, row, col, params, wfc, bfc):
    mesh = plsc.VectorSubcoreMesh(core_axis_name="c", subcore_axis_name="s")

    @functools.partial(
        pl.kernel,
        mesh=mesh,
        out_type=jax.ShapeDtypeStruct((6400,), jnp.float32),
        compiler_params=pltpu.CompilerParams(
            use_tc_tiling_on_sc=False, needs_layout_passes=False),
        scratch_types=[
            pltpu.VMEM((N_PAD,), jnp.float32),
            pltpu.VMEM((1504,), jnp.int32),
            pltpu.VMEM((1504,), jnp.int32),
            pltpu.VMEM((16,), jnp.float32),
            pltpu.VMEM((94, COLS_PT), jnp.float32),
            pltpu.VMEM((COLS_PT,), jnp.float32),
            pltpu.VMEM((N_PAD,), jnp.float32),
            pltpu.VMEM((N_PAD,), jnp.float32),
            pltpu.VMEM((N_PAD,), jnp.float32),
            pltpu.VMEM((N_PAD,), jnp.float32),
            pltpu.VMEM((N_PAD,), jnp.float32),
            pltpu.VMEM((N_PAD,), jnp.float32),
            pltpu.SemaphoreType.DMA,
        ],
    )
    def k(feat_hbm, row_hbm, col_hbm, par_hbm, wfc_hbm, bfc_hbm, out_hbm,
          feat_v, row_v, col_v, par_v, wblk_v, obuf_v, dinv_v,
          s1_v, g1_v, g2_v, agg2_v, v_v, sem):
        wid = lax.axis_index("s") * 2 + lax.axis_index("c")
        base = wid * COLS_PT

        wcp = pltpu.async_copy(wfc_hbm.at[:, pl.ds(base, COLS_PT)],
                               wblk_v, sem)
        pltpu.sync_copy(feat_hbm, feat_v)
        pltpu.sync_copy(row_hbm, row_v)
        pltpu.sync_copy(col_hbm, col_v)
        pltpu.sync_copy(par_hbm, par_v)
        pltpu.sync_copy(bfc_hbm.at[pl.ds(base, COLS_PT)], obuf_v)

        ones = jnp.ones((16,), jnp.float32)
        for i in range(N_PAD // 16):
            dinv_v[pl.ds(i * 16, 16)] = ones

        def deg_body(e, _):
            c = col_v[pl.ds(e * 16, 16)]
            plsc.addupdate_scatter(dinv_v, [c], ones)
            return 0

        lax.fori_loop(0, N_EDGE_CH, deg_body, 0, unroll=4)

        pv = par_v[pl.ds(0, 16)]
        w10, w11, w12, w13 = pv[0], pv[1], pv[2], pv[3]
        b10, b11, b12, b13 = pv[4], pv[5], pv[6], pv[7]
        w20, w21, w22, w23 = pv[8], pv[9], pv[10], pv[11]
        b2s = pv[12]

        for i in range(N_PAD // 16):
            sl = pl.ds(i * 16, 16)
            di = _rsqrt(dinv_v[sl])
            dinv_v[sl] = di
            g = di * feat_v[sl]
            g1_v[sl] = g
            s1_v[sl] = g

        def edge1_body(e, _):
            sl = pl.ds(e * 16, 16)
            r = row_v[sl]
            c = col_v[sl]
            g = plsc.load_gather(g1_v, [r])
            plsc.addupdate_scatter(s1_v, [c], g)
            return 0

        lax.fori_loop(0, N_EDGE_CH, edge1_body, 0, unroll=4)

        for i in range(N_PAD // 16):
            sl = pl.ds(i * 16, 16)
            di = dinv_v[sl]
            m = s1_v[sl] * di
            h0 = _tanh(m * w10 + b10)
            h1 = _tanh(m * w11 + b11)
            h2 = _tanh(m * w12 + b12)
            h3 = _tanh(m * w13 + b13)
            x2 = h0 * w20 + h1 * w21 + h2 * w22 + h3 * w23
            g2 = di * x2
            g2_v[sl] = g2
            agg2_v[sl] = g2

        def edge2_body(e, _):
            sl = pl.ds(e * 16, 16)
            r = row_v[sl]
            c = col_v[sl]
            g = plsc.load_gather(g2_v, [r])
            plsc.addupdate_scatter(agg2_v, [c], g)
            return 0

        lax.fori_loop(0, N_EDGE_CH, edge2_body, 0, unroll=4)

        for i in range(N_PAD // 16):
            sl = pl.ds(i * 16, 16)
            v_v[sl] = _tanh(agg2_v[sl] * dinv_v[sl] + b2s)

        wcp.wait()

        def mv_outer(i, accs):
            vvec = v_v[pl.ds(i * 16, 16)]
            nb = i * 16
            for l in range(16):
                s = vvec[l]
                accs = tuple(accs[j] + s * wblk_v[nb + l, pl.ds(OFFS[j], 16)]
                             for j in range(len(OFFS)))
            return accs

        init = tuple(obuf_v[pl.ds(o, 16)] for o in OFFS)
        accs = lax.fori_loop(0, 5, mv_outer, init)
        vtail = v_v[pl.ds(80, 16)]
        for l in range(14):
            s = vtail[l]
            accs = tuple(accs[j] + s * wblk_v[80 + l, pl.ds(OFFS[j], 16)]
                         for j in range(len(OFFS)))
        for j, o in enumerate(OFFS):
            obuf_v[pl.ds(o, 16)] = accs[j]
        pltpu.sync_copy(obuf_v, out_hbm.at[pl.ds(base, COLS_PT)])

    return k(feat, row, col, params, wfc, bfc)


def kernel(feature, edge_index, W1, b1, W2, b2, Wfc, bfc):
    feat = jnp.zeros((N_PAD,), jnp.float32).at[:94].set(feature[:, 0])
    row = edge_index[0].astype(jnp.int32)
    col = edge_index[1].astype(jnp.int32)
    params = jnp.concatenate([
        W1[0], b1, W2[:, 0], b2, jnp.zeros((3,), jnp.float32)])
    return _sc_gcn(feat, row, col, params, Wfc, bfc)


# speedup vs baseline: 3.5649x; 1.0802x over previous
import functools

import jax
import jax.numpy as jnp
from jax import lax
from jax.experimental import pallas as pl
from jax.experimental.pallas import tpu as pltpu
from jax.experimental.pallas import tpu_sc as plsc

N_PAD = 96
N_EDGE_CH = 94
NUM_CORES = 2
NUM_TILES = 16 * NUM_CORES
COLS_PT = 6400 // NUM_TILES
OFFS = tuple(list(range(0, COLS_PT - 15, 16))
             + ([COLS_PT - 16] if COLS_PT % 16 else []))


def _tanh(x):
    ax = jnp.abs(x)
    t = 1.0 - 2.0 / (jnp.exp(2.0 * ax) + 1.0)
    return jnp.sign(x) * t


def _rsqrt(d):
    bits = lax.bitcast_convert_type(d, jnp.int32)
    y = lax.bitcast_convert_type(
        jnp.int32(0x5F3759DF) - (bits >> 1), jnp.float32)
    half = 0.5 * d
    for _ in range(4):
        y = y * (1.5 - half * y * y)
    return y


def _sc_gcn(ed, fp, wfc, bfc):
    mesh = plsc.VectorSubcoreMesh(
        core_axis_name="c", subcore_axis_name="s", num_cores=NUM_CORES)

    @functools.partial(
        pl.kernel,
        mesh=mesh,
        out_type=jax.ShapeDtypeStruct((6400,), jnp.float32),
        compiler_params=pltpu.CompilerParams(
            use_tc_tiling_on_sc=False, needs_layout_passes=False),
        scratch_types=[
            pltpu.VMEM((3008,), jnp.int32),
            pltpu.VMEM((112,), jnp.float32),
            pltpu.VMEM((94, COLS_PT), jnp.float32),
            pltpu.VMEM((COLS_PT,), jnp.float32),
            pltpu.VMEM((N_PAD,), jnp.float32),
            pltpu.VMEM((N_PAD,), jnp.float32),
            pltpu.VMEM((N_PAD,), jnp.float32),
            pltpu.VMEM((N_PAD,), jnp.float32),
            pltpu.VMEM((N_PAD,), jnp.float32),
            pltpu.VMEM((N_PAD,), jnp.float32),
            pltpu.SemaphoreType.DMA,
            pltpu.SemaphoreType.DMA,
        ],
    )
    def k(ed_hbm, fp_hbm, wfc_hbm, bfc_hbm, out_hbm,
          ed_v, fp_v, wblk_v, obuf_v, dinv_v,
          s1_v, g1_v, g2_v, agg2_v, v_v, wsem, ssem):
        wid = lax.axis_index("s") * NUM_CORES + lax.axis_index("c")
        base = wid * COLS_PT

        wcp = pltpu.make_async_copy(
            wfc_hbm.at[:, pl.ds(base, COLS_PT)], wblk_v, wsem)
        wcp.start()
        cps = [
            pltpu.make_async_copy(ed_hbm, ed_v, ssem),
            pltpu.make_async_copy(fp_hbm, fp_v, ssem),
            pltpu.make_async_copy(
                bfc_hbm.at[pl.ds(base, COLS_PT)], obuf_v, ssem),
        ]
        for cp in cps:
            cp.start()
        for cp in cps:
            cp.wait()

        ones = jnp.ones((16,), jnp.float32)
        for i in range(N_PAD // 16):
            dinv_v[pl.ds(i * 16, 16)] = ones

        def deg_body(e, _):
            c = ed_v[pl.ds(1504 + e * 16, 16)]
            plsc.addupdate_scatter(dinv_v, [c], ones)
            return 0

        lax.fori_loop(0, N_EDGE_CH, deg_body, 0, unroll=4)

        pv = fp_v[pl.ds(96, 16)]
        w10, w11, w12, w13 = pv[0], pv[1], pv[2], pv[3]
        b10, b11, b12, b13 = pv[4], pv[5], pv[6], pv[7]
        w20, w21, w22, w23 = pv[8], pv[9], pv[10], pv[11]
        b2s = pv[12]

        for i in range(N_PAD // 16):
            sl = pl.ds(i * 16, 16)
            di = _rsqrt(dinv_v[sl])
            dinv_v[sl] = di
            g = di * fp_v[sl]
            g1_v[sl] = g
            s1_v[sl] = g

        def edge1_body(e, _):
            r = ed_v[pl.ds(e * 16, 16)]
            c = ed_v[pl.ds(1504 + e * 16, 16)]
            g = plsc.load_gather(g1_v, [r])
            plsc.addupdate_scatter(s1_v, [c], g)
            return 0

        lax.fori_loop(0, N_EDGE_CH, edge1_body, 0, unroll=4)

        for i in range(N_PAD // 16):
            sl = pl.ds(i * 16, 16)
            di = dinv_v[sl]
            m = s1_v[sl] * di
            h0 = _tanh(m * w10 + b10)
            h1 = _tanh(m * w11 + b11)
            h2 = _tanh(m * w12 + b12)
            h3 = _tanh(m * w13 + b13)
            x2 = h0 * w20 + h1 * w21 + h2 * w22 + h3 * w23
            g2 = di * x2
            g2_v[sl] = g2
            agg2_v[sl] = g2

        def edge2_body(e, _):
            r = ed_v[pl.ds(e * 16, 16)]
            c = ed_v[pl.ds(1504 + e * 16, 16)]
            g = plsc.load_gather(g2_v, [r])
            plsc.addupdate_scatter(agg2_v, [c], g)
            return 0

        lax.fori_loop(0, N_EDGE_CH, edge2_body, 0, unroll=4)

        for i in range(N_PAD // 16):
            sl = pl.ds(i * 16, 16)
            v_v[sl] = _tanh(agg2_v[sl] * dinv_v[sl] + b2s)

        wcp.wait()

        def mv_outer(i, accs):
            vvec = v_v[pl.ds(i * 16, 16)]
            nb = i * 16
            for l in range(16):
                s = vvec[l]
                accs = tuple(accs[j] + s * wblk_v[nb + l, pl.ds(OFFS[j], 16)]
                             for j in range(len(OFFS)))
            return accs

        init = tuple(obuf_v[pl.ds(o, 16)] for o in OFFS)
        accs = lax.fori_loop(0, 5, mv_outer, init)
        vtail = v_v[pl.ds(80, 16)]
        for l in range(14):
            s = vtail[l]
            accs = tuple(accs[j] + s * wblk_v[80 + l, pl.ds(OFFS[j], 16)]
                         for j in range(len(OFFS)))
        for j, o in enumerate(OFFS):
            obuf_v[pl.ds(o, 16)] = accs[j]
        pltpu.sync_copy(obuf_v, out_hbm.at[pl.ds(base, COLS_PT)])

    return k(ed, fp, wfc, bfc)


def kernel(feature, edge_index, W1, b1, W2, b2, Wfc, bfc):
    ed = edge_index.astype(jnp.int32).reshape(-1)
    feat = jnp.zeros((N_PAD,), jnp.float32).at[:94].set(feature[:, 0])
    params = jnp.concatenate([
        W1[0], b1, W2[:, 0], b2, jnp.zeros((3,), jnp.float32)])
    fp = jnp.concatenate([feat, params])
    return _sc_gcn(ed, fp, Wfc, bfc)
